# Initial kernel scaffold; baseline (speedup 1.0000x reference)
import jax
import jax.numpy as jnp
from jax.experimental import pallas as pl


def kernel(n_fans, n_shopkeepers, fan_factors, shopkeeper_factors):
    raise NotImplementedError("write your pallas kernel here")


# speedup vs baseline: 1.5650x; 1.5650x over previous
import functools

import jax
import jax.numpy as jnp
from jax.experimental import pallas as pl


def _tile_kernel(f_ref, s_ref, o_ref):
    logits = jnp.dot(f_ref[...], s_ref[...].T, preferred_element_type=jnp.float32)
    o_ref[...] = jax.nn.sigmoid(logits)


def _run(fan_factors, shopkeeper_factors):
    m, d = fan_factors.shape
    n = shopkeeper_factors.shape[0]
    bm = 512
    grid = (m // bm,)
    return pl.pallas_call(
        _tile_kernel,
        grid=grid,
        in_specs=[
            pl.BlockSpec((bm, d), lambda i: (i, 0)),
            pl.BlockSpec((n, d), lambda i: (0, 0)),
        ],
        out_specs=pl.BlockSpec((bm, n), lambda i: (i, 0)),
        out_shape=jax.ShapeDtypeStruct((m, n), jnp.float32),
    )(fan_factors, shopkeeper_factors)


def kernel(n_fans, n_shopkeepers, fan_factors, shopkeeper_factors):
    return _run(fan_factors, shopkeeper_factors)


# speedup vs baseline: 1.6111x; 1.0295x over previous
import functools

import jax
import jax.numpy as jnp
from jax.experimental import pallas as pl


def _tile_kernel(f_ref, s_ref, o_ref):
    logits = jnp.dot(f_ref[...], s_ref[...].T, preferred_element_type=jnp.float32)
    o_ref[...] = jax.nn.sigmoid(logits)


def _run(fan_factors, shopkeeper_factors):
    m, d = fan_factors.shape
    n = shopkeeper_factors.shape[0]
    bm = 1024
    grid = (m // bm,)
    return pl.pallas_call(
        _tile_kernel,
        grid=grid,
        in_specs=[
            pl.BlockSpec((bm, d), lambda i: (i, 0)),
            pl.BlockSpec((n, d), lambda i: (0, 0)),
        ],
        out_specs=pl.BlockSpec((bm, n), lambda i: (i, 0)),
        out_shape=jax.ShapeDtypeStruct((m, n), jnp.float32),
    )(fan_factors, shopkeeper_factors)


def kernel(n_fans, n_shopkeepers, fan_factors, shopkeeper_factors):
    return _run(fan_factors, shopkeeper_factors)
